# Initial kernel scaffold; baseline (speedup 1.0000x reference)
import jax
import jax.numpy as jnp
from jax.experimental import pallas as pl


def kernel(edge_index_r0, edge_index_r1, edge_index_r2, src, dst, emb, W1_r0, b1_r0, W1_r1, b1_r1, W1_r2, b1_r2, W2_r0, b2_r0, W2_r1, b2_r1, W2_r2, b2_r2):
    raise NotImplementedError("write your pallas kernel here")


# speedup vs baseline: 2.6860x; 2.6860x over previous
import functools

import jax
import jax.numpy as jnp
from jax import lax
from jax.experimental import pallas as pl
from jax.experimental.pallas import tpu as pltpu
from jax.experimental.pallas import tpu_sc as plsc

_N = 10000
_D = 128
_E = 100000
_B = 8192

_NC = 2
_NS = 16
_NW = _NC * _NS
_L = 16

_NPAD = 10240
_EPW = 3200
_EP = _EPW * _NW
_KCH = _EPW // 128
_RPT = _NPAD // _NS

@functools.cache
def _mesh():
    return plsc.VectorSubcoreMesh(core_axis_name="c", subcore_axis_name="s",
                                  num_cores=_NC, num_subcores=_NS)


def _zero_vmem(ref, nrows, ncols):
    z = jnp.zeros((_L,), jnp.float32)

    def body(i, _):
        for j in range(ncols // _L):
            ref[i, pl.ds(j * _L, _L)] = z
        return 0

    lax.fori_loop(0, nrows, body, 0)


@functools.cache
def _make_sc_degrees():
    return pl.kernel(
        _sc_degrees_body,
        out_type=jax.ShapeDtypeStruct((_NC, _NS, 6, _NPAD), jnp.float32),
        mesh=_mesh(),
        scratch_types=[pltpu.VMEM((_KCH, 128), jnp.int32)]
        + [pltpu.VMEM((_NPAD,), jnp.float32) for _ in range(6)],
        compiler_params=pltpu.CompilerParams(needs_layout_passes=False),
    )


def _sc_degrees_body(s0, s1, s2, d0, d1, d2, out, idx_v,
                     h0, h1, h2, h3, h4, h5):
    c = lax.axis_index("c")
    s = lax.axis_index("s")
    w = s * _NC + c
    hists = (h0, h1, h2, h3, h4, h5)

    zero = jnp.zeros((_L,), jnp.float32)
    ones = jnp.ones((_L,), jnp.float32)

    def zero_body(i, _):
        for h in hists:
            h[pl.ds(i * _L, _L)] = zero
        return 0

    lax.fori_loop(0, _NPAD // _L, zero_body, 0)

    for src_ref, h in zip((s0, s1, s2, d0, d1, d2), hists):
        pltpu.sync_copy(src_ref.at[w], idx_v)

        def grp(j, _):
            for k in range(8):
                iv = idx_v[j, pl.ds(k * _L, _L)]
                plsc.addupdate_scatter(h, [iv], ones)
            return 0

        lax.fori_loop(0, _KCH, grp, 0)

    for ai, h in enumerate(hists):
        pltpu.sync_copy(h, out.at[c, s, ai])


_NH = 1
_DH = _D // _NH


@functools.cache
def _make_sc_propagate():
    return pl.kernel(
        _sc_propagate_body,
        out_type=jax.ShapeDtypeStruct((_NC, 3, _NH, _NPAD, _DH),
                                      jnp.float32),
        mesh=_mesh(),
        scratch_types=[
            pltpu.VMEM((_KCH, 128), jnp.int32),
            pltpu.VMEM((_KCH, 128), jnp.int32),
            pltpu.VMEM((2, 128, _DH), jnp.float32),
            pltpu.VMEM((16, _DH), jnp.float32),
            pltpu.VMEM_SHARED((_NPAD, _DH), jnp.float32),
            pltpu.SemaphoreType.DMA,
        ],
    )


def _sc_propagate_body(x, s0, s1, s2, d0, d1, d2, out,
                       idx_s, idx_d, rows_v, zero_v, acc, sem):
    c = lax.axis_index("c")
    s = lax.axis_index("s")
    w = s * _NC + c

    _zero_vmem(zero_v, 16, _DH)

    for r, (sref, dref) in enumerate(zip((s0, s1, s2), (d0, d1, d2))):
        pltpu.sync_copy(sref.at[w], idx_s)
        pltpu.sync_copy(dref.at[w], idx_d)
        for h in range(_NH):
            table = x.at[r, h]
            def zero_blk(z, _):
                pltpu.sync_copy(zero_v, acc.at[pl.ds(s * _RPT + z * 16, 16)])
                return 0

            lax.fori_loop(0, _RPT // 16, zero_blk, 0)
            plsc.subcore_barrier()

            pltpu.async_copy(table.at[idx_s.at[0]], rows_v.at[0], sem)

            def chunk(j, _):
                p = lax.rem(j, 2)
                pltpu.make_async_copy(table.at[idx_s.at[j]], rows_v.at[p],
                                      sem).wait()

                @pl.when(j + 1 < _KCH)
                def _issue():
                    pltpu.async_copy(table.at[idx_s.at[j + 1]],
                                     rows_v.at[1 - p], sem)

                pltpu.sync_copy(rows_v.at[p], acc.at[idx_d.at[j]], add=True)
                return 0

            lax.fori_loop(0, _KCH, chunk, 0)
            plsc.subcore_barrier()
            pltpu.sync_copy(acc.at[pl.ds(s * _RPT, _RPT)],
                            out.at[c, r, h, pl.ds(s * _RPT, _RPT)])


@functools.cache
def _make_sc_gather_out():
    return pl.kernel(
        _sc_gather_out_body,
        out_type=(jax.ShapeDtypeStruct((_B, _D), jnp.float32),
                  jax.ShapeDtypeStruct((_B, _D), jnp.float32)),
        mesh=_mesh(),
        scratch_types=[
            pltpu.VMEM((_B // 128 // _NW, 128), jnp.int32),
            pltpu.VMEM((128, _D), jnp.float32),
            pltpu.SemaphoreType.DMA,
        ],
    )


def _sc_gather_out_body(h2, srcm, dstm, o1, o2, idx_v, rows_v, sem):
    c = lax.axis_index("c")
    s = lax.axis_index("s")
    w = s * _NC + c
    nrow = _B // 128 // _NW
    for idxm, out in ((srcm, o1), (dstm, o2)):
        pltpu.sync_copy(idxm.at[w], idx_v)
        for k in range(nrow):
            row = w * nrow + k
            pltpu.async_copy(h2.at[idx_v.at[k]], rows_v, sem).wait()
            pltpu.sync_copy(rows_v, out.at[pl.ds(row * 128, 128)])


_BLK = 512
_GRID = _NPAD // _BLK


def _tc1_body(emb_ref, w1_ref, degp_ref, x_ref, nout_ref, nin_ref):
    deg = jnp.sum(degp_ref[...], axis=(0, 1))
    nrm = lax.rsqrt(jnp.maximum(deg, 1.0))
    nout = nrm[0:3]
    nin = nrm[3:6]
    nout_ref[...] = nout
    nin_ref[...] = nin
    e = emb_ref[...]
    for r in range(3):
        res = jnp.dot(e, w1_ref[r], preferred_element_type=jnp.float32) \
            * nout[r][:, None]
        for hh in range(_NH):
            x_ref[r, hh] = res[:, hh * _DH:(hh + 1) * _DH]


def _tc_transform1(emb, w1, degp):
    return pl.pallas_call(
        _tc1_body,
        grid=(_GRID,),
        in_specs=[
            pl.BlockSpec((_BLK, _D), lambda i: (i, 0)),
            pl.BlockSpec((3, _D, _D), lambda i: (0, 0, 0)),
            pl.BlockSpec((_NC, _NS, 6, _BLK), lambda i: (0, 0, 0, i)),
        ],
        out_specs=[
            pl.BlockSpec((3, _NH, _BLK, _DH), lambda i: (0, 0, i, 0)),
            pl.BlockSpec((3, _BLK), lambda i: (0, i)),
            pl.BlockSpec((3, _BLK), lambda i: (0, i)),
        ],
        out_shape=[
            jax.ShapeDtypeStruct((3, _NH, _NPAD, _DH), jnp.float32),
            jax.ShapeDtypeStruct((3, _NPAD), jnp.float32),
            jax.ShapeDtypeStruct((3, _NPAD), jnp.float32),
        ],
    )(emb, w1, degp)


def _combine(p_ref, nin, bs_ref):
    halves = []
    for h in range(_NH):
        ph = bs_ref[...][:, h * _DH:(h + 1) * _DH] \
            * jnp.ones((_BLK, 1), jnp.float32)
        for r in range(3):
            ph = ph + (p_ref[0, r, h] + p_ref[1, r, h]) * nin[r][:, None]
        halves.append(ph)
    return jnp.tanh(jnp.concatenate(halves, axis=1))


def _tc2_body(p_ref, nin_ref, nout_ref, b1s_ref, w2_ref, x2_ref):
    nin = nin_ref[...]
    nout = nout_ref[...]
    h = _combine(p_ref, nin, b1s_ref)
    for r in range(3):
        res = jnp.dot(h, w2_ref[r], preferred_element_type=jnp.float32) \
            * nout[r][:, None]
        for hh in range(_NH):
            x2_ref[r, hh] = res[:, hh * _DH:(hh + 1) * _DH]


def _tc_combine_transform(p1, nin, nout, b1s, w2):
    return pl.pallas_call(
        _tc2_body,
        grid=(_GRID,),
        in_specs=[
            pl.BlockSpec((_NC, 3, _NH, _BLK, _DH), lambda i: (0, 0, 0, i, 0)),
            pl.BlockSpec((3, _BLK), lambda i: (0, i)),
            pl.BlockSpec((3, _BLK), lambda i: (0, i)),
            pl.BlockSpec((1, _D), lambda i: (0, 0)),
            pl.BlockSpec((3, _D, _D), lambda i: (0, 0, 0)),
        ],
        out_specs=pl.BlockSpec((3, _NH, _BLK, _DH), lambda i: (0, 0, i, 0)),
        out_shape=jax.ShapeDtypeStruct((3, _NH, _NPAD, _DH), jnp.float32),
    )(p1, nin, nout, b1s, w2)


def _tc3_body(p_ref, nin_ref, b2s_ref, h2_ref):
    h2_ref[...] = _combine(p_ref, nin_ref[...], b2s_ref)


def _tc_combine2(p2, nin, b2s):
    return pl.pallas_call(
        _tc3_body,
        grid=(_GRID,),
        in_specs=[
            pl.BlockSpec((_NC, 3, _NH, _BLK, _DH), lambda i: (0, 0, 0, i, 0)),
            pl.BlockSpec((3, _BLK), lambda i: (0, i)),
            pl.BlockSpec((1, _D), lambda i: (0, 0)),
        ],
        out_specs=pl.BlockSpec((_BLK, _D), lambda i: (i, 0)),
        out_shape=jax.ShapeDtypeStruct((_NPAD, _D), jnp.float32),
    )(p2, nin, b2s)


def _pad_edges(e):
    pad = jnp.full((_EP - _E,), _N, jnp.int32)
    srow = jnp.concatenate([e[0].astype(jnp.int32), pad]).reshape(
        _NW, _KCH, 128)
    drow = jnp.concatenate([e[1].astype(jnp.int32), pad]).reshape(
        _NW, _KCH, 128)
    return srow, drow


def kernel(edge_index_r0, edge_index_r1, edge_index_r2, src, dst, emb,
           W1_r0, b1_r0, W1_r1, b1_r1, W1_r2, b1_r2,
           W2_r0, b2_r0, W2_r1, b2_r1, W2_r2, b2_r2):
    s0, d0 = _pad_edges(edge_index_r0)
    s1, d1 = _pad_edges(edge_index_r1)
    s2, d2 = _pad_edges(edge_index_r2)
    emb_p = jnp.pad(emb.astype(jnp.float32), ((0, _NPAD - _N), (0, 0)))
    w1 = jnp.stack([W1_r0, W1_r1, W1_r2]).astype(jnp.float32)
    w2 = jnp.stack([W2_r0, W2_r1, W2_r2]).astype(jnp.float32)
    b1s = (b1_r0 + b1_r1 + b1_r2).astype(jnp.float32).reshape(1, _D)
    b2s = (b2_r0 + b2_r1 + b2_r2).astype(jnp.float32).reshape(1, _D)
    srcm = src.astype(jnp.int32).reshape(_NW, -1, 128)
    dstm = dst.astype(jnp.int32).reshape(_NW, -1, 128)

    sc_degrees = _make_sc_degrees()
    sc_propagate = _make_sc_propagate()
    sc_gather_out = _make_sc_gather_out()

    degp = sc_degrees(s0, s1, s2, d0, d1, d2)
    x1, nout, nin = _tc_transform1(emb_p, w1, degp)
    p1 = sc_propagate(x1, s0, s1, s2, d0, d1, d2)
    x2 = _tc_combine_transform(p1, nin, nout, b1s, w2)
    p2 = sc_propagate(x2, s0, s1, s2, d0, d1, d2)
    h2 = _tc_combine2(p2, nin, b2s)
    o1, o2 = sc_gather_out(h2, srcm, dstm)
    return (o1, o2)
